# baseline (speedup 1.0000x reference)
import functools
import jax
import jax.numpy as jnp
from jax import lax
from jax.experimental import pallas as pl
from jax.experimental.pallas import tpu as pltpu
from jax.experimental.pallas import tpu_sc as plsc

N, D = 2048, 768
E, DFF, H = 8, 1024, 12
DH = D // H
BLK = 256
NBLK = N // BLK
_EPS = 1e-5

_A = 2 * N
_TBLK = 128
_PADDED = _A + E * _TBLK
_MBLK = _PADDED // _TBLK
_BE_PAD = 48

_DNT = (((1,), (1,)), ((), ()))


def _ln(x, g, b):
    m = jnp.mean(x, axis=-1, keepdims=True)
    v = jnp.mean((x - m) ** 2, axis=-1, keepdims=True)
    return (x - m) * lax.rsqrt(v + _EPS) * g + b


def _gc_qkv_kernel(adj_ref, xfull_ref, xblk_ref, gcw_ref, gcb_ref, root_ref,
                   inw_ref, inb_ref, x1_ref, qkv_ref):
    g = jnp.dot(adj_ref[...], xfull_ref[...], preferred_element_type=jnp.float32)
    g = lax.dot_general(g, gcw_ref[...], _DNT, preferred_element_type=jnp.float32)
    g = g + gcb_ref[...] + root_ref[...]
    x1 = xblk_ref[...] + g
    x1_ref[...] = x1
    qkv = lax.dot_general(x1, inw_ref[...], _DNT, preferred_element_type=jnp.float32) + inb_ref[...]
    qkv_ref[...] = qkv


def _attn_kernel(qkvblk_ref, qkvfull_ref, out_ref):
    scale = 1.0 / (DH ** 0.5)
    for h in range(H):
        q = qkvblk_ref[:, h * DH:(h + 1) * DH]
        k = qkvfull_ref[:, D + h * DH:D + (h + 1) * DH]
        v = qkvfull_ref[:, 2 * D + h * DH:2 * D + (h + 1) * DH]
        s = lax.dot_general(q, k, _DNT, preferred_element_type=jnp.float32) * scale
        m = jnp.max(s, axis=-1, keepdims=True)
        p = jnp.exp(s - m)
        p = p / jnp.sum(p, axis=-1, keepdims=True)
        out_ref[:, h * DH:(h + 1) * DH] = jnp.dot(p, v, preferred_element_type=jnp.float32)


def _post_attn_kernel(attn_ref, x1_ref, outw_ref, outb_ref, ln1g_ref, ln1b_ref,
                      gatew_ref, x2_ref, e12_ref, w12_ref):
    a = lax.dot_general(attn_ref[...], outw_ref[...], _DNT,
                        preferred_element_type=jnp.float32) + outb_ref[...]
    pre = x1_ref[...] + a
    x2 = _ln(pre, ln1g_ref[...], ln1b_ref[...])
    x2_ref[...] = x2
    gl = lax.dot_general(x2, gatew_ref[...], _DNT, preferred_element_type=jnp.float32)
    l1 = jnp.max(gl, axis=-1, keepdims=True)
    iota = lax.broadcasted_iota(jnp.int32, gl.shape, 1)
    a1 = jnp.min(jnp.where(gl == l1, iota, E), axis=-1, keepdims=True)
    gl2 = jnp.where(iota == a1, -jnp.inf, gl)
    l2 = jnp.max(gl2, axis=-1, keepdims=True)
    a2 = jnp.min(jnp.where(gl2 == l2, iota, E), axis=-1, keepdims=True)
    w1 = 1.0 / (1.0 + jnp.exp(l2 - l1))
    e12_ref[...] = jnp.concatenate([a1, a2], axis=1)
    w12_ref[...] = jnp.concatenate([w1, 1.0 - w1], axis=1)


def _cumsum16(x):
    iota = lax.iota(jnp.int32, 16)
    y = x
    for k in (1, 2, 4, 8):
        sh = y.at[jnp.maximum(iota - k, 0)].get(mode="promise_in_bounds")
        y = y + jnp.where(iota >= k, sh, 0.0)
    return y


def _bcast_last(x):
    return x.at[jnp.zeros((16,), jnp.int32) + 15].get(mode="promise_in_bounds")


def _dispatch_kernel(e12_hbm, bexp_hbm, pos_hbm, e12_v, bexp_v, pos_v):
    wid = lax.axis_index("s") * 2 + lax.axis_index("c")

    @pl.when(wid == 0)
    def _():
        pltpu.sync_copy(e12_hbm, e12_v)

        def histbody(c, counts):
            v = e12_v[pl.ds(c * 16, 16)]
            return tuple(counts[e] + jnp.where(v == e, 1.0, 0.0)
                         for e in range(E))

        counts = lax.fori_loop(0, _A // 16, histbody,
                               tuple(jnp.zeros((16,), jnp.float32)
                                     for _ in range(E)))

        starts = []
        off = jnp.zeros((16,), jnp.int32)
        for e in range(E):
            starts.append(off)
            tot = _bcast_last(_cumsum16(counts[e])).astype(jnp.int32)
            off = off + lax.shift_left(
                lax.shift_right_logical(tot + (_TBLK - 1), 7), 7)

        for j in range(_BE_PAD // 16):
            bs = (lax.iota(jnp.int32, 16) + (16 * j)) * _TBLK
            be = jnp.zeros((16,), jnp.int32)
            for e in range(E):
                be = be + jnp.where(bs >= starts[e], 1, 0)
            bexp_v[pl.ds(j * 16, 16)] = be - 1

        def placebody(c, carry):
            v = e12_v[pl.ds(c * 16, 16)]
            pos = jnp.zeros((16,), jnp.int32)
            newc = []
            for e in range(E):
                m = v == e
                onesf = jnp.where(m, 1.0, 0.0)
                cs = _cumsum16(onesf)
                base = starts[e] + carry[e].astype(jnp.int32)
                pos = pos + jnp.where(m, cs.astype(jnp.int32) - 1 + base, 0)
                newc.append(carry[e] + _bcast_last(cs))
            pos_v[pl.ds(c * 16, 16)] = pos
            return tuple(newc)

        lax.fori_loop(0, _A // 16, placebody,
                      tuple(jnp.zeros((16,), jnp.float32) for _ in range(E)))

        pltpu.sync_copy(bexp_v, bexp_hbm)
        pltpu.sync_copy(pos_v, pos_hbm)


def _dispatch(e12_flat):
    mesh = plsc.VectorSubcoreMesh(core_axis_name="c", subcore_axis_name="s")
    return pl.kernel(
        _dispatch_kernel,
        out_type=[jax.ShapeDtypeStruct((_BE_PAD,), jnp.int32),
                  jax.ShapeDtypeStruct((_A,), jnp.int32)],
        mesh=mesh,
        scratch_types=[pltpu.VMEM((_A,), jnp.int32),
                       pltpu.VMEM((_BE_PAD,), jnp.int32),
                       pltpu.VMEM((_A,), jnp.int32)],
    )(e12_flat)


def _scatter_rows_kernel(bpw, x2_hbm, pos_hbm, xs_hbm,
                         tok1, tok2, pos1, pos2, rows1, rows2,
                         g1, g2, s1, s2):
    wid = lax.axis_index("s") * 2 + lax.axis_index("c")
    base = wid * bpw
    half = bpw // 2

    def idxbody(j, carry):
        tok1[pl.ds(j * 16, 16)] = lax.shift_right_logical(
            lax.iota(jnp.int32, 16) + (base + j * 16), 1)
        tok2[pl.ds(j * 16, 16)] = lax.shift_right_logical(
            lax.iota(jnp.int32, 16) + (base + half + j * 16), 1)
        return carry

    lax.fori_loop(0, half // 16, idxbody, jnp.int32(0))
    pltpu.sync_copy(pos_hbm.at[pl.ds(base, half)], pos1)
    pltpu.sync_copy(pos_hbm.at[pl.ds(base + half, half)], pos2)
    c1 = pltpu.async_copy(x2_hbm.at[tok1], rows1, g1)
    c2 = pltpu.async_copy(x2_hbm.at[tok2], rows2, g2)
    c1.wait()
    w1 = pltpu.async_copy(rows1, xs_hbm.at[pos1], s1)
    c2.wait()
    w2 = pltpu.async_copy(rows2, xs_hbm.at[pos2], s2)
    w1.wait()
    w2.wait()


def _scatter_rows(x2, pos):
    info = plsc.get_sparse_core_info()
    nw = info.num_cores * info.num_subcores
    bpw = _A // nw
    half = bpw // 2
    mesh = plsc.VectorSubcoreMesh(core_axis_name="c", subcore_axis_name="s")
    return pl.kernel(
        functools.partial(_scatter_rows_kernel, bpw),
        out_type=jax.ShapeDtypeStruct((_PADDED, D), jnp.float32),
        mesh=mesh,
        scratch_types=[pltpu.VMEM((half,), jnp.int32),
                       pltpu.VMEM((half,), jnp.int32),
                       pltpu.VMEM((half,), jnp.int32),
                       pltpu.VMEM((half,), jnp.int32),
                       pltpu.VMEM((half, D), jnp.float32),
                       pltpu.VMEM((half, D), jnp.float32),
                       pltpu.SemaphoreType.DMA,
                       pltpu.SemaphoreType.DMA,
                       pltpu.SemaphoreType.DMA,
                       pltpu.SemaphoreType.DMA],
    )(x2, pos)


def _gather_rows_kernel(bpw, table_hbm, idx_hbm, out_hbm, idx_v, rows_v, sem):
    wid = lax.axis_index("s") * 2 + lax.axis_index("c")
    base = wid * bpw
    pltpu.sync_copy(idx_hbm.at[pl.ds(base, bpw)], idx_v)
    pltpu.async_copy(table_hbm.at[idx_v], rows_v, sem).wait()
    pltpu.sync_copy(rows_v, out_hbm.at[pl.ds(base, bpw)])


def _gather_rows(table, idx, nrows):
    info = plsc.get_sparse_core_info()
    nw = info.num_cores * info.num_subcores
    bpw = nrows // nw
    mesh = plsc.VectorSubcoreMesh(core_axis_name="c", subcore_axis_name="s")
    return pl.kernel(
        functools.partial(_gather_rows_kernel, bpw),
        out_type=jax.ShapeDtypeStruct((nrows, D), jnp.float32),
        mesh=mesh,
        scratch_types=[pltpu.VMEM((bpw,), jnp.int32),
                       pltpu.VMEM((bpw, D), jnp.float32),
                       pltpu.SemaphoreType.DMA],
    )(table, idx)


def _moe_kernel(bexp_ref, xs_ref, w1_ref, b1_ref, w2_ref, b2_ref, ys_ref):
    i = pl.program_id(0)
    e = bexp_ref[i]
    xb = xs_ref[...].astype(jnp.bfloat16)
    h = lax.dot_general(xb, w1_ref[e], _DNT,
                        preferred_element_type=jnp.float32) + b1_ref[e]
    h = 0.5 * h * (1.0 + lax.erf(h * (2.0 ** -0.5)))
    ys_ref[...] = lax.dot_general(h.astype(jnp.bfloat16), w2_ref[e], _DNT,
                                  preferred_element_type=jnp.float32) + b2_ref[e]


def _combine_kernel(y12_ref, w12_ref, x2_ref, ln2g_ref, ln2b_ref, out_ref):
    w = w12_ref[...]
    moe = w[:, 0:1] * y12_ref[:, :D] + w[:, 1:2] * y12_ref[:, D:]
    out_ref[...] = _ln(x2_ref[...] + moe, ln2g_ref[...], ln2b_ref[...])


def kernel(x, adj, gc_W, gc_b, root_emb, in_proj_w, in_proj_b, out_proj_w,
           out_proj_b, ln1_g, ln1_b, ln2_g, ln2_b, gate_w, e_w1, e_b1, e_w2, e_b2):
    xf = x.reshape(N, D)
    adjf = adj.reshape(N, N)
    row = lambda a: a.reshape(1, -1)

    x1, qkv = pl.pallas_call(
        _gc_qkv_kernel,
        grid=(NBLK,),
        in_specs=[
            pl.BlockSpec((BLK, N), lambda i: (i, 0)),
            pl.BlockSpec((N, D), lambda i: (0, 0)),
            pl.BlockSpec((BLK, D), lambda i: (i, 0)),
            pl.BlockSpec((D, D), lambda i: (0, 0)),
            pl.BlockSpec((1, D), lambda i: (0, 0)),
            pl.BlockSpec((1, D), lambda i: (0, 0)),
            pl.BlockSpec((3 * D, D), lambda i: (0, 0)),
            pl.BlockSpec((1, 3 * D), lambda i: (0, 0)),
        ],
        out_specs=[pl.BlockSpec((BLK, D), lambda i: (i, 0)),
                   pl.BlockSpec((BLK, 3 * D), lambda i: (i, 0))],
        out_shape=[jax.ShapeDtypeStruct((N, D), jnp.float32),
                   jax.ShapeDtypeStruct((N, 3 * D), jnp.float32)],
    )(adjf, xf, xf, gc_W, row(gc_b), root_emb, in_proj_w, row(in_proj_b))

    attn = pl.pallas_call(
        _attn_kernel,
        grid=(NBLK,),
        in_specs=[
            pl.BlockSpec((BLK, 3 * D), lambda i: (i, 0)),
            pl.BlockSpec((N, 3 * D), lambda i: (0, 0)),
        ],
        out_specs=pl.BlockSpec((BLK, D), lambda i: (i, 0)),
        out_shape=jax.ShapeDtypeStruct((N, D), jnp.float32),
    )(qkv, qkv)

    x2, e12, w12 = pl.pallas_call(
        _post_attn_kernel,
        grid=(NBLK,),
        in_specs=[
            pl.BlockSpec((BLK, D), lambda i: (i, 0)),
            pl.BlockSpec((BLK, D), lambda i: (i, 0)),
            pl.BlockSpec((D, D), lambda i: (0, 0)),
            pl.BlockSpec((1, D), lambda i: (0, 0)),
            pl.BlockSpec((1, D), lambda i: (0, 0)),
            pl.BlockSpec((1, D), lambda i: (0, 0)),
            pl.BlockSpec((E, D), lambda i: (0, 0)),
        ],
        out_specs=[pl.BlockSpec((BLK, D), lambda i: (i, 0)),
                   pl.BlockSpec((BLK, 2), lambda i: (i, 0)),
                   pl.BlockSpec((BLK, 2), lambda i: (i, 0))],
        out_shape=[jax.ShapeDtypeStruct((N, D), jnp.float32),
                   jax.ShapeDtypeStruct((N, 2), jnp.int32),
                   jax.ShapeDtypeStruct((N, 2), jnp.float32)],
    )(attn, x1, out_proj_w, row(out_proj_b), row(ln1_g), row(ln1_b), gate_w)
    block_expert, pos = _dispatch(e12.reshape(_A))

    x_sorted = _scatter_rows(x2, pos)

    ys = pl.pallas_call(
        _moe_kernel,
        grid=(_MBLK,),
        in_specs=[
            pl.BlockSpec(memory_space=pltpu.SMEM),
            pl.BlockSpec((_TBLK, D), lambda i: (i, 0)),
            pl.BlockSpec((E, DFF, D), lambda i: (0, 0, 0)),
            pl.BlockSpec((E, 1, DFF), lambda i: (0, 0, 0)),
            pl.BlockSpec((E, D, DFF), lambda i: (0, 0, 0)),
            pl.BlockSpec((E, 1, D), lambda i: (0, 0, 0)),
        ],
        out_specs=pl.BlockSpec((_TBLK, D), lambda i: (i, 0)),
        out_shape=jax.ShapeDtypeStruct((_PADDED, D), jnp.float32),
    )(block_expert, x_sorted, e_w1.astype(jnp.bfloat16), e_b1.reshape(E, 1, DFF),
      e_w2.astype(jnp.bfloat16), e_b2.reshape(E, 1, D))

    y12 = _gather_rows(ys, pos, _A).reshape(N, 2 * D)

    out = pl.pallas_call(
        _combine_kernel,
        grid=(NBLK,),
        in_specs=[
            pl.BlockSpec((BLK, 2 * D), lambda i: (i, 0)),
            pl.BlockSpec((BLK, 2), lambda i: (i, 0)),
            pl.BlockSpec((BLK, D), lambda i: (i, 0)),
            pl.BlockSpec((1, D), lambda i: (0, 0)),
            pl.BlockSpec((1, D), lambda i: (0, 0)),
        ],
        out_specs=pl.BlockSpec((BLK, D), lambda i: (i, 0)),
        out_shape=jax.ShapeDtypeStruct((N, D), jnp.float32),
    )(y12, w12, x2, row(ln2_g), row(ln2_b))

    return out.reshape(x.shape)


# speedup vs baseline: 1.0452x; 1.0452x over previous
import functools
import jax
import jax.numpy as jnp
from jax import lax
from jax.experimental import pallas as pl
from jax.experimental.pallas import tpu as pltpu
from jax.experimental.pallas import tpu_sc as plsc

N, D = 2048, 768
E, DFF, H = 8, 1024, 12
DH = D // H
BLK = 256
NBLK = N // BLK
_EPS = 1e-5

_A = 2 * N
_TBLK = 128
_PADDED = _A + E * _TBLK
_MBLK = _PADDED // _TBLK
_BE_PAD = 48

_DNT = (((1,), (1,)), ((), ()))


def _ln(x, g, b):
    m = jnp.mean(x, axis=-1, keepdims=True)
    v = jnp.mean((x - m) ** 2, axis=-1, keepdims=True)
    return (x - m) * lax.rsqrt(v + _EPS) * g + b


def _gc_qkv_kernel(adj_ref, xfull_ref, xblk_ref, gcw_ref, gcb_ref, root_ref,
                   inw_ref, inb_ref, x1_ref, qkv_ref):
    g = jnp.dot(adj_ref[...], xfull_ref[...], preferred_element_type=jnp.float32)
    g = lax.dot_general(g, gcw_ref[...], _DNT, preferred_element_type=jnp.float32)
    g = g + gcb_ref[...] + root_ref[...]
    x1 = xblk_ref[...] + g
    x1_ref[...] = x1
    qkv = lax.dot_general(x1, inw_ref[...], _DNT, preferred_element_type=jnp.float32) + inb_ref[...]
    qkv_ref[...] = qkv


def _attn_kernel(qkvblk_ref, qkvfull_ref, out_ref):
    scale = 1.0 / (DH ** 0.5)
    for h in range(H):
        q = qkvblk_ref[:, h * DH:(h + 1) * DH]
        k = qkvfull_ref[:, D + h * DH:D + (h + 1) * DH]
        v = qkvfull_ref[:, 2 * D + h * DH:2 * D + (h + 1) * DH]
        s = lax.dot_general(q, k, _DNT, preferred_element_type=jnp.float32) * scale
        m = jnp.max(s, axis=-1, keepdims=True)
        p = jnp.exp(s - m)
        p = p / jnp.sum(p, axis=-1, keepdims=True)
        out_ref[:, h * DH:(h + 1) * DH] = jnp.dot(p, v, preferred_element_type=jnp.float32)


def _post_attn_kernel(attn_ref, x1_ref, outw_ref, outb_ref, ln1g_ref, ln1b_ref,
                      gatew_ref, x2_ref, e12_ref, w12_ref):
    a = lax.dot_general(attn_ref[...], outw_ref[...], _DNT,
                        preferred_element_type=jnp.float32) + outb_ref[...]
    pre = x1_ref[...] + a
    x2 = _ln(pre, ln1g_ref[...], ln1b_ref[...])
    x2_ref[...] = x2
    gl = lax.dot_general(x2, gatew_ref[...], _DNT, preferred_element_type=jnp.float32)
    l1 = jnp.max(gl, axis=-1, keepdims=True)
    iota = lax.broadcasted_iota(jnp.int32, gl.shape, 1)
    a1 = jnp.min(jnp.where(gl == l1, iota, E), axis=-1, keepdims=True)
    gl2 = jnp.where(iota == a1, -jnp.inf, gl)
    l2 = jnp.max(gl2, axis=-1, keepdims=True)
    a2 = jnp.min(jnp.where(gl2 == l2, iota, E), axis=-1, keepdims=True)
    w1 = 1.0 / (1.0 + jnp.exp(l2 - l1))
    e12_ref[...] = jnp.concatenate([a1, a2], axis=1)
    w12_ref[...] = jnp.concatenate([w1, 1.0 - w1], axis=1)


def _cumsum16(x):
    iota = lax.iota(jnp.int32, 16)
    y = x
    for k in (1, 2, 4, 8):
        sh = y.at[jnp.maximum(iota - k, 0)].get(mode="promise_in_bounds")
        y = y + jnp.where(iota >= k, sh, 0.0)
    return y


def _bcast_last(x):
    return x.at[jnp.zeros((16,), jnp.int32) + 15].get(mode="promise_in_bounds")


def _dispatch_kernel(e12_hbm, bexp_hbm, pos_hbm, e12_v, bexp_v, pos_v):
    wid = lax.axis_index("s") * 2 + lax.axis_index("c")

    @pl.when(wid == 0)
    def _():
        pltpu.sync_copy(e12_hbm, e12_v)

        def histbody(c, counts):
            v = e12_v[pl.ds(c * 16, 16)]
            return tuple(counts[e] + jnp.where(v == e, 1.0, 0.0)
                         for e in range(E))

        counts = lax.fori_loop(0, _A // 16, histbody,
                               tuple(jnp.zeros((16,), jnp.float32)
                                     for _ in range(E)))

        starts = []
        off = jnp.zeros((16,), jnp.int32)
        for e in range(E):
            starts.append(off)
            tot = _bcast_last(_cumsum16(counts[e])).astype(jnp.int32)
            off = off + lax.shift_left(
                lax.shift_right_logical(tot + (_TBLK - 1), 7), 7)

        for j in range(_BE_PAD // 16):
            bs = (lax.iota(jnp.int32, 16) + (16 * j)) * _TBLK
            be = jnp.zeros((16,), jnp.int32)
            for e in range(E):
                be = be + jnp.where(bs >= starts[e], 1, 0)
            bexp_v[pl.ds(j * 16, 16)] = be - 1

        def placebody(c, carry):
            v = e12_v[pl.ds(c * 16, 16)]
            pos = jnp.zeros((16,), jnp.int32)
            newc = []
            for e in range(E):
                m = v == e
                onesf = jnp.where(m, 1.0, 0.0)
                cs = _cumsum16(onesf)
                base = starts[e] + carry[e].astype(jnp.int32)
                pos = pos + jnp.where(m, cs.astype(jnp.int32) - 1 + base, 0)
                newc.append(carry[e] + _bcast_last(cs))
            pos_v[pl.ds(c * 16, 16)] = pos
            return tuple(newc)

        lax.fori_loop(0, _A // 16, placebody,
                      tuple(jnp.zeros((16,), jnp.float32) for _ in range(E)))

        pltpu.sync_copy(bexp_v, bexp_hbm)
        pltpu.sync_copy(pos_v, pos_hbm)


def _dispatch(e12_flat):
    mesh = plsc.VectorSubcoreMesh(core_axis_name="c", subcore_axis_name="s")
    return pl.kernel(
        _dispatch_kernel,
        out_type=[jax.ShapeDtypeStruct((_BE_PAD,), jnp.int32),
                  jax.ShapeDtypeStruct((_A,), jnp.int32)],
        mesh=mesh,
        scratch_types=[pltpu.VMEM((_A,), jnp.int32),
                       pltpu.VMEM((_BE_PAD,), jnp.int32),
                       pltpu.VMEM((_A,), jnp.int32)],
    )(e12_flat)


def _scatter_rows_kernel(bpw, x2_hbm, pos_hbm, xs_hbm,
                         tok1, tok2, pos1, pos2, rows1, rows2,
                         g1, g2, s1, s2):
    wid = lax.axis_index("s") * 2 + lax.axis_index("c")
    base = wid * bpw
    half = bpw // 2

    def idxbody(j, carry):
        tok1[pl.ds(j * 16, 16)] = lax.shift_right_logical(
            lax.iota(jnp.int32, 16) + (base + j * 16), 1)
        tok2[pl.ds(j * 16, 16)] = lax.shift_right_logical(
            lax.iota(jnp.int32, 16) + (base + half + j * 16), 1)
        return carry

    lax.fori_loop(0, half // 16, idxbody, jnp.int32(0))
    pltpu.sync_copy(pos_hbm.at[pl.ds(base, half)], pos1)
    pltpu.sync_copy(pos_hbm.at[pl.ds(base + half, half)], pos2)
    c1 = pltpu.async_copy(x2_hbm.at[tok1], rows1, g1)
    c2 = pltpu.async_copy(x2_hbm.at[tok2], rows2, g2)
    c1.wait()
    w1 = pltpu.async_copy(rows1, xs_hbm.at[pos1], s1)
    c2.wait()
    w2 = pltpu.async_copy(rows2, xs_hbm.at[pos2], s2)
    w1.wait()
    w2.wait()


def _scatter_rows(x2, pos):
    info = plsc.get_sparse_core_info()
    nw = info.num_cores * info.num_subcores
    bpw = _A // nw
    half = bpw // 2
    mesh = plsc.VectorSubcoreMesh(core_axis_name="c", subcore_axis_name="s")
    return pl.kernel(
        functools.partial(_scatter_rows_kernel, bpw),
        out_type=jax.ShapeDtypeStruct((_PADDED, D), jnp.float32),
        mesh=mesh,
        scratch_types=[pltpu.VMEM((half,), jnp.int32),
                       pltpu.VMEM((half,), jnp.int32),
                       pltpu.VMEM((half,), jnp.int32),
                       pltpu.VMEM((half,), jnp.int32),
                       pltpu.VMEM((half, D), jnp.float32),
                       pltpu.VMEM((half, D), jnp.float32),
                       pltpu.SemaphoreType.DMA,
                       pltpu.SemaphoreType.DMA,
                       pltpu.SemaphoreType.DMA,
                       pltpu.SemaphoreType.DMA],
    )(x2, pos)


def _gather_rows_kernel(bpw, table_hbm, idx_hbm, out_hbm, idx_v, rows_v, sem):
    wid = lax.axis_index("s") * 2 + lax.axis_index("c")
    base = wid * bpw
    pltpu.sync_copy(idx_hbm.at[pl.ds(base, bpw)], idx_v)
    pltpu.async_copy(table_hbm.at[idx_v], rows_v, sem).wait()
    pltpu.sync_copy(rows_v, out_hbm.at[pl.ds(base, bpw)])


def _gather_rows(table, idx, nrows):
    info = plsc.get_sparse_core_info()
    nw = info.num_cores * info.num_subcores
    bpw = nrows // nw
    mesh = plsc.VectorSubcoreMesh(core_axis_name="c", subcore_axis_name="s")
    return pl.kernel(
        functools.partial(_gather_rows_kernel, bpw),
        out_type=jax.ShapeDtypeStruct((nrows, D), jnp.float32),
        mesh=mesh,
        scratch_types=[pltpu.VMEM((bpw,), jnp.int32),
                       pltpu.VMEM((bpw, D), jnp.float32),
                       pltpu.SemaphoreType.DMA],
    )(table, idx)


def _moe_kernel(bexp_ref, xs_ref, w1_ref, b1_ref, w2_ref, b2_ref, ys_ref):
    h = lax.dot_general(xs_ref[...], w1_ref[0], _DNT,
                        preferred_element_type=jnp.float32) + b1_ref[0]
    h = 0.5 * h * (1.0 + lax.erf(h * (2.0 ** -0.5)))
    ys_ref[...] = lax.dot_general(h, w2_ref[0], _DNT,
                                  preferred_element_type=jnp.float32) + b2_ref[0]


def _combine_kernel(y12_ref, w12_ref, x2_ref, ln2g_ref, ln2b_ref, out_ref):
    w = w12_ref[...]
    moe = w[:, 0:1] * y12_ref[:, :D] + w[:, 1:2] * y12_ref[:, D:]
    out_ref[...] = _ln(x2_ref[...] + moe, ln2g_ref[...], ln2b_ref[...])


def kernel(x, adj, gc_W, gc_b, root_emb, in_proj_w, in_proj_b, out_proj_w,
           out_proj_b, ln1_g, ln1_b, ln2_g, ln2_b, gate_w, e_w1, e_b1, e_w2, e_b2):
    xf = x.reshape(N, D)
    adjf = adj.reshape(N, N)
    row = lambda a: a.reshape(1, -1)

    x1, qkv = pl.pallas_call(
        _gc_qkv_kernel,
        grid=(NBLK,),
        in_specs=[
            pl.BlockSpec((BLK, N), lambda i: (i, 0)),
            pl.BlockSpec((N, D), lambda i: (0, 0)),
            pl.BlockSpec((BLK, D), lambda i: (i, 0)),
            pl.BlockSpec((D, D), lambda i: (0, 0)),
            pl.BlockSpec((1, D), lambda i: (0, 0)),
            pl.BlockSpec((1, D), lambda i: (0, 0)),
            pl.BlockSpec((3 * D, D), lambda i: (0, 0)),
            pl.BlockSpec((1, 3 * D), lambda i: (0, 0)),
        ],
        out_specs=[pl.BlockSpec((BLK, D), lambda i: (i, 0)),
                   pl.BlockSpec((BLK, 3 * D), lambda i: (i, 0))],
        out_shape=[jax.ShapeDtypeStruct((N, D), jnp.float32),
                   jax.ShapeDtypeStruct((N, 3 * D), jnp.float32)],
    )(adjf, xf, xf, gc_W, row(gc_b), root_emb, in_proj_w, row(in_proj_b))

    attn = pl.pallas_call(
        _attn_kernel,
        grid=(NBLK,),
        in_specs=[
            pl.BlockSpec((BLK, 3 * D), lambda i: (i, 0)),
            pl.BlockSpec((N, 3 * D), lambda i: (0, 0)),
        ],
        out_specs=pl.BlockSpec((BLK, D), lambda i: (i, 0)),
        out_shape=jax.ShapeDtypeStruct((N, D), jnp.float32),
    )(qkv, qkv)

    x2, e12, w12 = pl.pallas_call(
        _post_attn_kernel,
        grid=(NBLK,),
        in_specs=[
            pl.BlockSpec((BLK, D), lambda i: (i, 0)),
            pl.BlockSpec((BLK, D), lambda i: (i, 0)),
            pl.BlockSpec((D, D), lambda i: (0, 0)),
            pl.BlockSpec((1, D), lambda i: (0, 0)),
            pl.BlockSpec((1, D), lambda i: (0, 0)),
            pl.BlockSpec((1, D), lambda i: (0, 0)),
            pl.BlockSpec((E, D), lambda i: (0, 0)),
        ],
        out_specs=[pl.BlockSpec((BLK, D), lambda i: (i, 0)),
                   pl.BlockSpec((BLK, 2), lambda i: (i, 0)),
                   pl.BlockSpec((BLK, 2), lambda i: (i, 0))],
        out_shape=[jax.ShapeDtypeStruct((N, D), jnp.float32),
                   jax.ShapeDtypeStruct((N, 2), jnp.int32),
                   jax.ShapeDtypeStruct((N, 2), jnp.float32)],
    )(attn, x1, out_proj_w, row(out_proj_b), row(ln1_g), row(ln1_b), gate_w)
    block_expert, pos = _dispatch(e12.reshape(_A))

    x_sorted = _scatter_rows(x2, pos)

    ys = pl.pallas_call(
        _moe_kernel,
        grid_spec=pltpu.PrefetchScalarGridSpec(
            num_scalar_prefetch=1,
            grid=(_MBLK,),
            in_specs=[
                pl.BlockSpec((_TBLK, D), lambda i, b: (i, 0)),
                pl.BlockSpec((1, DFF, D), lambda i, b: (b[i], 0, 0)),
                pl.BlockSpec((1, 1, DFF), lambda i, b: (b[i], 0, 0)),
                pl.BlockSpec((1, D, DFF), lambda i, b: (b[i], 0, 0)),
                pl.BlockSpec((1, 1, D), lambda i, b: (b[i], 0, 0)),
            ],
            out_specs=pl.BlockSpec((_TBLK, D), lambda i, b: (i, 0)),
        ),
        out_shape=jax.ShapeDtypeStruct((_PADDED, D), jnp.float32),
    )(block_expert, x_sorted, e_w1, e_b1.reshape(E, 1, DFF),
      e_w2, e_b2.reshape(E, 1, D))

    y12 = _gather_rows(ys, pos, _A).reshape(N, 2 * D)

    out = pl.pallas_call(
        _combine_kernel,
        grid=(NBLK,),
        in_specs=[
            pl.BlockSpec((BLK, 2 * D), lambda i: (i, 0)),
            pl.BlockSpec((BLK, 2), lambda i: (i, 0)),
            pl.BlockSpec((BLK, D), lambda i: (i, 0)),
            pl.BlockSpec((1, D), lambda i: (0, 0)),
            pl.BlockSpec((1, D), lambda i: (0, 0)),
        ],
        out_specs=pl.BlockSpec((BLK, D), lambda i: (i, 0)),
        out_shape=jax.ShapeDtypeStruct((N, D), jnp.float32),
    )(y12, w12, x2, row(ln2_g), row(ln2_b))

    return out.reshape(x.shape)


# speedup vs baseline: 1.1609x; 1.1107x over previous
import functools
import jax
import jax.numpy as jnp
from jax import lax
from jax.experimental import pallas as pl
from jax.experimental.pallas import tpu as pltpu
from jax.experimental.pallas import tpu_sc as plsc

N, D = 2048, 768
E, DFF, H = 8, 1024, 12
DH = D // H
BLK = 256
NBLK = N // BLK
_EPS = 1e-5

_A = 2 * N
_TBLK = 128
_PADDED = _A + E * _TBLK
_MBLK = _PADDED // _TBLK
_BE_PAD = 48

_DNT = (((1,), (1,)), ((), ()))


def _ln(x, g, b):
    m = jnp.mean(x, axis=-1, keepdims=True)
    v = jnp.mean((x - m) ** 2, axis=-1, keepdims=True)
    return (x - m) * lax.rsqrt(v + _EPS) * g + b


def _gc_qkv_kernel(adj_ref, xfull_ref, xblk_ref, gcw_ref, gcb_ref, root_ref,
                   inw_ref, inb_ref, x1_ref, qkv_ref):
    g = jnp.dot(adj_ref[...], xfull_ref[...], preferred_element_type=jnp.float32)
    g = lax.dot_general(g, gcw_ref[...], _DNT, preferred_element_type=jnp.float32)
    g = g + gcb_ref[...] + root_ref[...]
    x1 = xblk_ref[...] + g
    x1_ref[...] = x1
    qkv = lax.dot_general(x1, inw_ref[...], _DNT, preferred_element_type=jnp.float32) + inb_ref[...]
    scale = 1.0 / (DH ** 0.5)
    qkv_ref[:, :D] = qkv[:, :D] * scale
    qkv_ref[:, D:] = qkv[:, D:]


def _attn_kernel(qkvblk_ref, qkvfull_ref, out_ref):
    for h in range(H):
        q = qkvblk_ref[:, h * DH:(h + 1) * DH]
        k = qkvfull_ref[:, D + h * DH:D + (h + 1) * DH]
        v = qkvfull_ref[:, 2 * D + h * DH:2 * D + (h + 1) * DH]
        s = lax.dot_general(q, k, _DNT, preferred_element_type=jnp.float32)
        m = jnp.max(s, axis=-1, keepdims=True)
        p = jnp.exp(s - m)
        denom = jnp.sum(p, axis=-1, keepdims=True)
        o = jnp.dot(p, v, preferred_element_type=jnp.float32)
        out_ref[:, h * DH:(h + 1) * DH] = o / denom


def _post_attn_kernel(attn_ref, x1_ref, outw_ref, outb_ref, ln1g_ref, ln1b_ref,
                      gatew_ref, x2_ref, e12_ref, w12_ref):
    a = lax.dot_general(attn_ref[...], outw_ref[...], _DNT,
                        preferred_element_type=jnp.float32) + outb_ref[...]
    pre = x1_ref[...] + a
    x2 = _ln(pre, ln1g_ref[...], ln1b_ref[...])
    x2_ref[...] = x2
    gl = lax.dot_general(x2, gatew_ref[...], _DNT, preferred_element_type=jnp.float32)
    l1 = jnp.max(gl, axis=-1, keepdims=True)
    iota = lax.broadcasted_iota(jnp.int32, gl.shape, 1)
    a1 = jnp.min(jnp.where(gl == l1, iota, E), axis=-1, keepdims=True)
    gl2 = jnp.where(iota == a1, -jnp.inf, gl)
    l2 = jnp.max(gl2, axis=-1, keepdims=True)
    a2 = jnp.min(jnp.where(gl2 == l2, iota, E), axis=-1, keepdims=True)
    w1 = 1.0 / (1.0 + jnp.exp(l2 - l1))
    e12_ref[...] = jnp.concatenate([a1, a2], axis=1)
    w12_ref[...] = jnp.concatenate([w1, 1.0 - w1], axis=1)


def _cumsum16(x):
    iota = lax.iota(jnp.int32, 16)
    y = x
    for k in (1, 2, 4, 8):
        sh = y.at[jnp.maximum(iota - k, 0)].get(mode="promise_in_bounds")
        y = y + jnp.where(iota >= k, sh, 0.0)
    return y


def _bcast_last(x):
    return x.at[jnp.zeros((16,), jnp.int32) + 15].get(mode="promise_in_bounds")


def _dispatch_kernel(e12_hbm, bexp_hbm, pos_hbm, e12_v, bexp_v, pos_v):
    wid = lax.axis_index("s") * 2 + lax.axis_index("c")

    @pl.when(wid == 0)
    def _():
        pltpu.sync_copy(e12_hbm, e12_v)

        def histbody(c, counts):
            v = e12_v[pl.ds(c * 16, 16)]
            return tuple(counts[e] + jnp.where(v == e, 1.0, 0.0)
                         for e in range(E))

        counts = lax.fori_loop(0, _A // 16, histbody,
                               tuple(jnp.zeros((16,), jnp.float32)
                                     for _ in range(E)))

        starts = []
        off = jnp.zeros((16,), jnp.int32)
        for e in range(E):
            starts.append(off)
            tot = _bcast_last(_cumsum16(counts[e])).astype(jnp.int32)
            off = off + lax.shift_left(
                lax.shift_right_logical(tot + (_TBLK - 1), 7), 7)

        for j in range(_BE_PAD // 16):
            bs = (lax.iota(jnp.int32, 16) + (16 * j)) * _TBLK
            be = jnp.zeros((16,), jnp.int32)
            for e in range(E):
                be = be + jnp.where(bs >= starts[e], 1, 0)
            bexp_v[pl.ds(j * 16, 16)] = be - 1

        def placebody(c, carry):
            v = e12_v[pl.ds(c * 16, 16)]
            pos = jnp.zeros((16,), jnp.int32)
            newc = []
            for e in range(E):
                m = v == e
                onesf = jnp.where(m, 1.0, 0.0)
                cs = _cumsum16(onesf)
                base = starts[e] + carry[e].astype(jnp.int32)
                pos = pos + jnp.where(m, cs.astype(jnp.int32) - 1 + base, 0)
                newc.append(carry[e] + _bcast_last(cs))
            pos_v[pl.ds(c * 16, 16)] = pos
            return tuple(newc)

        lax.fori_loop(0, _A // 16, placebody,
                      tuple(jnp.zeros((16,), jnp.float32) for _ in range(E)))

        pltpu.sync_copy(bexp_v, bexp_hbm)
        pltpu.sync_copy(pos_v, pos_hbm)


def _dispatch(e12_flat):
    mesh = plsc.VectorSubcoreMesh(core_axis_name="c", subcore_axis_name="s")
    return pl.kernel(
        _dispatch_kernel,
        out_type=[jax.ShapeDtypeStruct((_BE_PAD,), jnp.int32),
                  jax.ShapeDtypeStruct((_A,), jnp.int32)],
        mesh=mesh,
        scratch_types=[pltpu.VMEM((_A,), jnp.int32),
                       pltpu.VMEM((_BE_PAD,), jnp.int32),
                       pltpu.VMEM((_A,), jnp.int32)],
    )(e12_flat)


def _scatter_rows_kernel(bpw, x2_hbm, pos_hbm, xs_hbm,
                         tok1, tok2, pos1, pos2, rows1, rows2,
                         g1, g2, s1, s2):
    wid = lax.axis_index("s") * 2 + lax.axis_index("c")
    base = wid * bpw
    half = bpw // 2

    def idxbody(j, carry):
        tok1[pl.ds(j * 16, 16)] = lax.shift_right_logical(
            lax.iota(jnp.int32, 16) + (base + j * 16), 1)
        tok2[pl.ds(j * 16, 16)] = lax.shift_right_logical(
            lax.iota(jnp.int32, 16) + (base + half + j * 16), 1)
        return carry

    lax.fori_loop(0, half // 16, idxbody, jnp.int32(0))
    pltpu.sync_copy(pos_hbm.at[pl.ds(base, half)], pos1)
    pltpu.sync_copy(pos_hbm.at[pl.ds(base + half, half)], pos2)
    c1 = pltpu.async_copy(x2_hbm.at[tok1], rows1, g1)
    c2 = pltpu.async_copy(x2_hbm.at[tok2], rows2, g2)
    c1.wait()
    w1 = pltpu.async_copy(rows1, xs_hbm.at[pos1], s1)
    c2.wait()
    w2 = pltpu.async_copy(rows2, xs_hbm.at[pos2], s2)
    w1.wait()
    w2.wait()


def _scatter_rows(x2, pos):
    info = plsc.get_sparse_core_info()
    nw = info.num_cores * info.num_subcores
    bpw = _A // nw
    half = bpw // 2
    mesh = plsc.VectorSubcoreMesh(core_axis_name="c", subcore_axis_name="s")
    return pl.kernel(
        functools.partial(_scatter_rows_kernel, bpw),
        out_type=jax.ShapeDtypeStruct((_PADDED, D), jnp.float32),
        mesh=mesh,
        scratch_types=[pltpu.VMEM((half,), jnp.int32),
                       pltpu.VMEM((half,), jnp.int32),
                       pltpu.VMEM((half,), jnp.int32),
                       pltpu.VMEM((half,), jnp.int32),
                       pltpu.VMEM((half, D), jnp.float32),
                       pltpu.VMEM((half, D), jnp.float32),
                       pltpu.SemaphoreType.DMA,
                       pltpu.SemaphoreType.DMA,
                       pltpu.SemaphoreType.DMA,
                       pltpu.SemaphoreType.DMA],
    )(x2, pos)


def _gather_rows_kernel(bpw, table_hbm, idx_hbm, out_hbm, idx_v, rows_v, sem):
    wid = lax.axis_index("s") * 2 + lax.axis_index("c")
    base = wid * bpw
    pltpu.sync_copy(idx_hbm.at[pl.ds(base, bpw)], idx_v)
    pltpu.async_copy(table_hbm.at[idx_v], rows_v, sem).wait()
    pltpu.sync_copy(rows_v, out_hbm.at[pl.ds(base, bpw)])


def _gather_rows(table, idx, nrows):
    info = plsc.get_sparse_core_info()
    nw = info.num_cores * info.num_subcores
    bpw = nrows // nw
    mesh = plsc.VectorSubcoreMesh(core_axis_name="c", subcore_axis_name="s")
    return pl.kernel(
        functools.partial(_gather_rows_kernel, bpw),
        out_type=jax.ShapeDtypeStruct((nrows, D), jnp.float32),
        mesh=mesh,
        scratch_types=[pltpu.VMEM((bpw,), jnp.int32),
                       pltpu.VMEM((bpw, D), jnp.float32),
                       pltpu.SemaphoreType.DMA],
    )(table, idx)


def _moe_kernel(bexp_ref, xs_ref, w1_ref, b1_ref, w2_ref, b2_ref, ys_ref):
    h = lax.dot_general(xs_ref[...], w1_ref[0], _DNT,
                        preferred_element_type=jnp.float32) + b1_ref[0]
    h = 0.5 * h * (1.0 + lax.erf(h * (2.0 ** -0.5)))
    ys_ref[...] = lax.dot_general(h, w2_ref[0], _DNT,
                                  preferred_element_type=jnp.float32) + b2_ref[0]


def _combine_kernel(y12_ref, w12_ref, x2_ref, ln2g_ref, ln2b_ref, out_ref):
    w = w12_ref[...]
    moe = w[:, 0:1] * y12_ref[:, :D] + w[:, 1:2] * y12_ref[:, D:]
    out_ref[...] = _ln(x2_ref[...] + moe, ln2g_ref[...], ln2b_ref[...])


def kernel(x, adj, gc_W, gc_b, root_emb, in_proj_w, in_proj_b, out_proj_w,
           out_proj_b, ln1_g, ln1_b, ln2_g, ln2_b, gate_w, e_w1, e_b1, e_w2, e_b2):
    xf = x.reshape(N, D)
    adjf = adj.reshape(N, N)
    row = lambda a: a.reshape(1, -1)

    x1, qkv = pl.pallas_call(
        _gc_qkv_kernel,
        grid=(NBLK,),
        in_specs=[
            pl.BlockSpec((BLK, N), lambda i: (i, 0)),
            pl.BlockSpec((N, D), lambda i: (0, 0)),
            pl.BlockSpec((BLK, D), lambda i: (i, 0)),
            pl.BlockSpec((D, D), lambda i: (0, 0)),
            pl.BlockSpec((1, D), lambda i: (0, 0)),
            pl.BlockSpec((1, D), lambda i: (0, 0)),
            pl.BlockSpec((3 * D, D), lambda i: (0, 0)),
            pl.BlockSpec((1, 3 * D), lambda i: (0, 0)),
        ],
        out_specs=[pl.BlockSpec((BLK, D), lambda i: (i, 0)),
                   pl.BlockSpec((BLK, 3 * D), lambda i: (i, 0))],
        out_shape=[jax.ShapeDtypeStruct((N, D), jnp.float32),
                   jax.ShapeDtypeStruct((N, 3 * D), jnp.float32)],
    )(adjf, xf, xf, gc_W, row(gc_b), root_emb, in_proj_w, row(in_proj_b))

    attn = pl.pallas_call(
        _attn_kernel,
        grid=(NBLK,),
        in_specs=[
            pl.BlockSpec((BLK, 3 * D), lambda i: (i, 0)),
            pl.BlockSpec((N, 3 * D), lambda i: (0, 0)),
        ],
        out_specs=pl.BlockSpec((BLK, D), lambda i: (i, 0)),
        out_shape=jax.ShapeDtypeStruct((N, D), jnp.float32),
    )(qkv, qkv)

    x2, e12, w12 = pl.pallas_call(
        _post_attn_kernel,
        grid=(NBLK,),
        in_specs=[
            pl.BlockSpec((BLK, D), lambda i: (i, 0)),
            pl.BlockSpec((BLK, D), lambda i: (i, 0)),
            pl.BlockSpec((D, D), lambda i: (0, 0)),
            pl.BlockSpec((1, D), lambda i: (0, 0)),
            pl.BlockSpec((1, D), lambda i: (0, 0)),
            pl.BlockSpec((1, D), lambda i: (0, 0)),
            pl.BlockSpec((E, D), lambda i: (0, 0)),
        ],
        out_specs=[pl.BlockSpec((BLK, D), lambda i: (i, 0)),
                   pl.BlockSpec((BLK, 2), lambda i: (i, 0)),
                   pl.BlockSpec((BLK, 2), lambda i: (i, 0))],
        out_shape=[jax.ShapeDtypeStruct((N, D), jnp.float32),
                   jax.ShapeDtypeStruct((N, 2), jnp.int32),
                   jax.ShapeDtypeStruct((N, 2), jnp.float32)],
    )(attn, x1, out_proj_w, row(out_proj_b), row(ln1_g), row(ln1_b), gate_w)
    block_expert, pos = _dispatch(e12.reshape(_A))

    x_sorted = _scatter_rows(x2, pos)

    ys = pl.pallas_call(
        _moe_kernel,
        grid_spec=pltpu.PrefetchScalarGridSpec(
            num_scalar_prefetch=1,
            grid=(_MBLK,),
            in_specs=[
                pl.BlockSpec((_TBLK, D), lambda i, b: (i, 0)),
                pl.BlockSpec((1, DFF, D), lambda i, b: (b[i], 0, 0)),
                pl.BlockSpec((1, 1, DFF), lambda i, b: (b[i], 0, 0)),
                pl.BlockSpec((1, D, DFF), lambda i, b: (b[i], 0, 0)),
                pl.BlockSpec((1, 1, D), lambda i, b: (b[i], 0, 0)),
            ],
            out_specs=pl.BlockSpec((_TBLK, D), lambda i, b: (i, 0)),
        ),
        out_shape=jax.ShapeDtypeStruct((_PADDED, D), jnp.float32),
    )(block_expert, x_sorted, e_w1, e_b1.reshape(E, 1, DFF),
      e_w2, e_b2.reshape(E, 1, D))

    y12 = _gather_rows(ys, pos, _A).reshape(N, 2 * D)

    out = pl.pallas_call(
        _combine_kernel,
        grid=(NBLK,),
        in_specs=[
            pl.BlockSpec((BLK, 2 * D), lambda i: (i, 0)),
            pl.BlockSpec((BLK, 2), lambda i: (i, 0)),
            pl.BlockSpec((BLK, D), lambda i: (i, 0)),
            pl.BlockSpec((1, D), lambda i: (0, 0)),
            pl.BlockSpec((1, D), lambda i: (0, 0)),
        ],
        out_specs=pl.BlockSpec((BLK, D), lambda i: (i, 0)),
        out_shape=jax.ShapeDtypeStruct((N, D), jnp.float32),
    )(y12, w12, x2, row(ln2_g), row(ln2_b))

    return out.reshape(x.shape)
